# baseline (speedup 1.0000x reference)
import functools

import jax
import jax.numpy as jnp
from jax import lax
from jax.experimental import pallas as pl
from jax.experimental.pallas import tpu as pltpu
from jax.experimental.pallas import tpu_sc as plsc

N_CTX = 4
N_CLS_CTX = 4
CTX_DIM = 512
CONTEXT_LEN = 77
PREFIX_LEN = N_CTX + 1
SUFFIX_LEN = CONTEXT_LEN - PREFIX_LEN - N_CLS_CTX
ROW = N_CLS_CTX * CTX_DIM
PRE_F = PREFIX_LEN * CTX_DIM
SUF_F = SUFFIX_LEN * CTX_DIM
OUT_F = CONTEXT_LEN * CTX_DIM


def _make_sc_gather(num_class: int, b: int):
    info = plsc.get_sparse_core_info()
    nc, ns = info.num_cores, info.num_subcores
    nw = nc * ns
    assert b % nw == 0 and (b // nw) % 8 == 0
    b_per_w = b // nw
    mesh = plsc.VectorSubcoreMesh(core_axis_name="c", subcore_axis_name="s")

    @functools.partial(
        pl.kernel,
        mesh=mesh,
        out_type=jax.ShapeDtypeStruct((b, ROW), jnp.float32),
        scratch_types=[
            pltpu.VMEM((b_per_w,), jnp.int32),
            pltpu.VMEM((b_per_w, ROW), jnp.float32),
            pltpu.SemaphoreType.DMA,
        ],
    )
    def gather(table_hbm, idx_hbm, out_hbm, idx_v, rows_v, sem):
        wid = lax.axis_index("s") * nc + lax.axis_index("c")
        base = wid * b_per_w
        pltpu.sync_copy(idx_hbm.at[pl.ds(base, b_per_w)], idx_v)
        pltpu.async_copy(table_hbm.at[idx_v], rows_v, sem).wait()
        pltpu.sync_copy(rows_v, out_hbm.at[pl.ds(base, b_per_w)])

    return gather


def _assemble_body(ctx_ref, pre_ref, suf_ref, out_ref):
    bb = out_ref.shape[0]
    out_ref[:, :PRE_F] = jnp.broadcast_to(pre_ref[...], (bb, PRE_F))
    out_ref[:, PRE_F:PRE_F + ROW] = ctx_ref[...]
    out_ref[:, PRE_F + ROW:] = jnp.broadcast_to(suf_ref[...], (bb, SUF_F))


def _make_tc_assemble(b: int, bb: int):
    grid = (b // bb,)
    return pl.pallas_call(
        _assemble_body,
        grid=grid,
        in_specs=[
            pl.BlockSpec((bb, ROW), lambda i: (i, 0)),
            pl.BlockSpec((1, PRE_F), lambda i: (0, 0)),
            pl.BlockSpec((1, SUF_F), lambda i: (0, 0)),
        ],
        out_specs=pl.BlockSpec((bb, OUT_F), lambda i: (i, 0)),
        out_shape=jax.ShapeDtypeStruct((b, OUT_F), jnp.float32),
    )


def kernel(label, cls_ctx, token_prefix, token_suffix):
    b = label.shape[0]
    num_class = cls_ctx.shape[0]
    table = cls_ctx.reshape(num_class, ROW)
    idx = label.astype(jnp.int32)
    ctx = _make_sc_gather(num_class, b)(table, idx)
    out = _make_tc_assemble(b, 8)(
        ctx,
        token_prefix.reshape(1, PRE_F),
        token_suffix.reshape(1, SUF_F),
    )
    return out.reshape(b, CONTEXT_LEN, CTX_DIM)


# speedup vs baseline: 4.1681x; 4.1681x over previous
import functools

import jax
import jax.numpy as jnp
from jax import lax
from jax.experimental import pallas as pl
from jax.experimental.pallas import tpu as pltpu
from jax.experimental.pallas import tpu_sc as plsc

N_CTX = 4
N_CLS_CTX = 4
CTX_DIM = 512
CONTEXT_LEN = 77
PREFIX_LEN = N_CTX + 1
SUFFIX_LEN = CONTEXT_LEN - PREFIX_LEN - N_CLS_CTX
ROW = N_CLS_CTX * CTX_DIM
PRE_F = PREFIX_LEN * CTX_DIM
SUF_F = SUFFIX_LEN * CTX_DIM
OUT_F = CONTEXT_LEN * CTX_DIM


def _make_sc_gather(num_class: int, b: int):
    info = plsc.get_sparse_core_info()
    nc, ns = info.num_cores, info.num_subcores
    nw = nc * ns
    assert b % nw == 0 and (b // nw) % 8 == 0
    b_per_w = b // nw
    mesh = plsc.VectorSubcoreMesh(core_axis_name="c", subcore_axis_name="s")

    @functools.partial(
        pl.kernel,
        mesh=mesh,
        out_type=jax.ShapeDtypeStruct((b, N_CLS_CTX, CTX_DIM), jnp.float32),
        scratch_types=[
            pltpu.VMEM((b_per_w,), jnp.int32),
            pltpu.VMEM((b_per_w, N_CLS_CTX, CTX_DIM), jnp.float32),
            pltpu.SemaphoreType.DMA,
        ],
    )
    def gather(table_hbm, idx_hbm, out_hbm, idx_v, rows_v, sem):
        wid = lax.axis_index("s") * nc + lax.axis_index("c")
        base = wid * b_per_w
        pltpu.sync_copy(idx_hbm.at[pl.ds(base, b_per_w)], idx_v)
        pltpu.async_copy(table_hbm.at[idx_v], rows_v, sem).wait()
        pltpu.sync_copy(rows_v, out_hbm.at[pl.ds(base, b_per_w)])

    return gather


def _assemble_body(ctx_ref, pre_ref, suf_ref, out_ref):
    bb = out_ref.shape[0]
    out_ref[:, :PREFIX_LEN, :] = jnp.broadcast_to(
        pre_ref[...], (bb, PREFIX_LEN, CTX_DIM))
    out_ref[:, PREFIX_LEN:PREFIX_LEN + N_CLS_CTX, :] = ctx_ref[...]
    out_ref[:, PREFIX_LEN + N_CLS_CTX:, :] = jnp.broadcast_to(
        suf_ref[...], (bb, SUFFIX_LEN, CTX_DIM))


def _make_tc_assemble(b: int, bb: int):
    grid = (b // bb,)
    return pl.pallas_call(
        _assemble_body,
        grid=grid,
        in_specs=[
            pl.BlockSpec((bb, N_CLS_CTX, CTX_DIM), lambda i: (i, 0, 0)),
            pl.BlockSpec((1, PREFIX_LEN, CTX_DIM), lambda i: (0, 0, 0)),
            pl.BlockSpec((1, SUFFIX_LEN, CTX_DIM), lambda i: (0, 0, 0)),
        ],
        out_specs=pl.BlockSpec((bb, CONTEXT_LEN, CTX_DIM), lambda i: (i, 0, 0)),
        out_shape=jax.ShapeDtypeStruct((b, CONTEXT_LEN, CTX_DIM), jnp.float32),
    )


def kernel(label, cls_ctx, token_prefix, token_suffix):
    b = label.shape[0]
    num_class = cls_ctx.shape[0]
    idx = label.astype(jnp.int32)
    ctx = _make_sc_gather(num_class, b)(cls_ctx, idx)
    out = _make_tc_assemble(b, 8)(ctx, token_prefix, token_suffix)
    return out


# speedup vs baseline: 5.1655x; 1.2393x over previous
import functools

import jax
import jax.numpy as jnp
from jax import lax
from jax.experimental import pallas as pl
from jax.experimental.pallas import tpu as pltpu
from jax.experimental.pallas import tpu_sc as plsc

N_CTX = 4
N_CLS_CTX = 4
CTX_DIM = 512
CONTEXT_LEN = 77
PREFIX_LEN = N_CTX + 1
SUFFIX_LEN = CONTEXT_LEN - PREFIX_LEN - N_CLS_CTX
ROW = N_CLS_CTX * CTX_DIM
PRE_F = PREFIX_LEN * CTX_DIM
SUF_F = SUFFIX_LEN * CTX_DIM
OUT_F = CONTEXT_LEN * CTX_DIM


def _make_sc_gather(num_class: int, b: int):
    info = plsc.get_sparse_core_info()
    nc, ns = info.num_cores, info.num_subcores
    nw = nc * ns
    assert b % nw == 0 and (b // nw) % 8 == 0
    b_per_w = b // nw
    mesh = plsc.VectorSubcoreMesh(core_axis_name="c", subcore_axis_name="s")

    @functools.partial(
        pl.kernel,
        mesh=mesh,
        out_type=jax.ShapeDtypeStruct((b, N_CLS_CTX, CTX_DIM), jnp.float32),
        scratch_types=[
            pltpu.VMEM((b_per_w,), jnp.int32),
            pltpu.VMEM((b_per_w, N_CLS_CTX, CTX_DIM), jnp.float32),
            pltpu.SemaphoreType.DMA,
        ],
    )
    def gather(table_hbm, idx_hbm, out_hbm, idx_v, rows_v, sem):
        wid = lax.axis_index("s") * nc + lax.axis_index("c")
        base = wid * b_per_w
        pltpu.sync_copy(idx_hbm.at[pl.ds(base, b_per_w)], idx_v)
        pltpu.async_copy(table_hbm.at[idx_v], rows_v, sem).wait()
        pltpu.sync_copy(rows_v, out_hbm.at[pl.ds(base, b_per_w)])

    return gather


def _assemble_body(ctx_ref, pre_ref, suf_ref, out_ref):
    bb = out_ref.shape[0]
    out_ref[:, :PREFIX_LEN, :] = jnp.broadcast_to(
        pre_ref[...], (bb, PREFIX_LEN, CTX_DIM))
    out_ref[:, PREFIX_LEN:PREFIX_LEN + N_CLS_CTX, :] = ctx_ref[...]
    out_ref[:, PREFIX_LEN + N_CLS_CTX:, :] = jnp.broadcast_to(
        suf_ref[...], (bb, SUFFIX_LEN, CTX_DIM))


def _make_tc_assemble(b: int, bb: int):
    grid = (b // bb,)
    return pl.pallas_call(
        _assemble_body,
        grid=grid,
        in_specs=[
            pl.BlockSpec((bb, N_CLS_CTX, CTX_DIM), lambda i: (i, 0, 0)),
            pl.BlockSpec((1, PREFIX_LEN, CTX_DIM), lambda i: (0, 0, 0)),
            pl.BlockSpec((1, SUFFIX_LEN, CTX_DIM), lambda i: (0, 0, 0)),
        ],
        out_specs=pl.BlockSpec((bb, CONTEXT_LEN, CTX_DIM), lambda i: (i, 0, 0)),
        out_shape=jax.ShapeDtypeStruct((b, CONTEXT_LEN, CTX_DIM), jnp.float32),
    )


def kernel(label, cls_ctx, token_prefix, token_suffix):
    b = label.shape[0]
    num_class = cls_ctx.shape[0]
    idx = label.astype(jnp.int32)
    ctx = _make_sc_gather(num_class, b)(cls_ctx, idx)
    out = _make_tc_assemble(b, 32)(ctx, token_prefix, token_suffix)
    return out


# speedup vs baseline: 5.2052x; 1.0077x over previous
import functools

import jax
import jax.numpy as jnp
from jax import lax
from jax.experimental import pallas as pl
from jax.experimental.pallas import tpu as pltpu
from jax.experimental.pallas import tpu_sc as plsc

N_CTX = 4
N_CLS_CTX = 4
CTX_DIM = 512
CONTEXT_LEN = 77
PREFIX_LEN = N_CTX + 1
SUFFIX_LEN = CONTEXT_LEN - PREFIX_LEN - N_CLS_CTX
ROW = N_CLS_CTX * CTX_DIM
PRE_F = PREFIX_LEN * CTX_DIM
SUF_F = SUFFIX_LEN * CTX_DIM
OUT_F = CONTEXT_LEN * CTX_DIM


def _make_sc_gather(num_class: int, b: int):
    info = plsc.get_sparse_core_info()
    nc, ns = info.num_cores, info.num_subcores
    nw = nc * ns
    assert b % nw == 0 and (b // nw) % 8 == 0
    b_per_w = b // nw
    mesh = plsc.VectorSubcoreMesh(core_axis_name="c", subcore_axis_name="s")

    @functools.partial(
        pl.kernel,
        mesh=mesh,
        out_type=jax.ShapeDtypeStruct((b, N_CLS_CTX, CTX_DIM), jnp.float32),
        scratch_types=[
            pltpu.VMEM((b_per_w,), jnp.int32),
            pltpu.VMEM((b_per_w, N_CLS_CTX, CTX_DIM), jnp.float32),
            pltpu.SemaphoreType.DMA,
        ],
    )
    def gather(table_hbm, idx_hbm, out_hbm, idx_v, rows_v, sem):
        wid = lax.axis_index("s") * nc + lax.axis_index("c")
        base = wid * b_per_w
        pltpu.sync_copy(idx_hbm.at[pl.ds(base, b_per_w)], idx_v)
        pltpu.async_copy(table_hbm.at[idx_v], rows_v, sem).wait()
        pltpu.sync_copy(rows_v, out_hbm.at[pl.ds(base, b_per_w)])

    return gather


def _assemble_body(ctx_ref, pre_ref, suf_ref, out_ref):
    bb = out_ref.shape[0]
    out_ref[:, :PREFIX_LEN, :] = jnp.broadcast_to(
        pre_ref[...], (bb, PREFIX_LEN, CTX_DIM))
    out_ref[:, PREFIX_LEN:PREFIX_LEN + N_CLS_CTX, :] = ctx_ref[...]
    out_ref[:, PREFIX_LEN + N_CLS_CTX:, :] = jnp.broadcast_to(
        suf_ref[...], (bb, SUFFIX_LEN, CTX_DIM))


def _make_tc_assemble(b: int, bb: int):
    grid = (b // bb,)
    return pl.pallas_call(
        _assemble_body,
        grid=grid,
        in_specs=[
            pl.BlockSpec((bb, N_CLS_CTX, CTX_DIM), lambda i: (i, 0, 0)),
            pl.BlockSpec((1, PREFIX_LEN, CTX_DIM), lambda i: (0, 0, 0)),
            pl.BlockSpec((1, SUFFIX_LEN, CTX_DIM), lambda i: (0, 0, 0)),
        ],
        out_specs=pl.BlockSpec((bb, CONTEXT_LEN, CTX_DIM), lambda i: (i, 0, 0)),
        out_shape=jax.ShapeDtypeStruct((b, CONTEXT_LEN, CTX_DIM), jnp.float32),
    )


def kernel(label, cls_ctx, token_prefix, token_suffix):
    b = label.shape[0]
    num_class = cls_ctx.shape[0]
    idx = label.astype(jnp.int32)
    ctx = _make_sc_gather(num_class, b)(cls_ctx, idx)
    out = _make_tc_assemble(b, 64)(ctx, token_prefix, token_suffix)
    return out
